# baseline (speedup 1.0000x reference)
import functools

import jax
import jax.numpy as jnp
from jax import lax
from jax.experimental import pallas as pl
from jax.experimental.pallas import tpu as pltpu
from jax.experimental.pallas import tpu_sc as plsc

B = 16384
C = 1001
NADV = 1000
S = 1.0 / NADV

NC, NS, L = 2, 16, 16
NW = NC * NS
PW = B // NW
CH = 32
NCH = PW // CH
NG = CH // L


def _make_sc():
    mesh = plsc.VectorSubcoreMesh(core_axis_name="c", subcore_axis_name="s")

    @functools.partial(
        pl.kernel,
        out_type=jax.ShapeDtypeStruct((B,), jnp.float32),
        mesh=mesh,
        compiler_params=pltpu.CompilerParams(needs_layout_passes=False),
        scratch_types=[
            pltpu.VMEM((CH, C), jnp.float32),
            pltpu.VMEM((CH, C), jnp.float32),
            pltpu.VMEM((PW,), jnp.int32),
            pltpu.VMEM((PW,), jnp.float32),
            pltpu.SemaphoreType.DMA((2,)),
            pltpu.SemaphoreType.DMA,
        ],
    )
    def sc_duelqa(x_hbm, a_hbm, out_hbm, xv0, xv1, av, ov, sems, asem):
        wid = lax.axis_index("s") * NC + lax.axis_index("c")
        base = wid * PW
        pltpu.async_copy(a_hbm.at[pl.ds(base, PW)], av, asem).wait()
        lane = lax.iota(jnp.int32, L)
        zero16 = jnp.zeros((L,), jnp.float32)
        tailm = (lane >= 8).astype(jnp.float32)

        def cp(c, b):
            return pltpu.make_async_copy(
                x_hbm.at[pl.ds(base + c * CH, CH), :],
                xv0 if b == 0 else xv1,
                sems.at[b],
            )

        def start_chunk(c, b):
            cp(c, b).start()

        def wait_chunk(c, b):
            cp(c, b).wait()

        start_chunk(0, 0)
        start_chunk(1, 1)

        def _chunk(c, b):
            wait_chunk(c, b)
            xb = xv0 if b == 0 else xv1
            for g in range(NG):
                lrows = g * L + lane

                def row_body(r, tvec):
                    row = g * L + r
                    accs = [zero16] * 8
                    for j in range(62):
                        accs[j % 8] = accs[j % 8] + xb[row, pl.ds(j * L, L)]
                    tail = xb[row, pl.ds(984, L)] * tailm
                    acc = (((accs[0] + accs[1]) + (accs[2] + accs[3]))
                           + ((accs[4] + accs[5]) + (accs[6] + accs[7]))) + tail
                    t = jnp.sum(acc)
                    oh = (lane == r).astype(jnp.float32)
                    return tvec + t * oh

                tvec = lax.fori_loop(0, L, row_body, zero16, unroll=2)
                off = c * CH + g * L
                a16 = av[pl.ds(off, L)]
                gv = plsc.load_gather(xb, [lrows, a16])
                ov[pl.ds(off, L)] = gv - tvec * jnp.float32(S)

            @pl.when(c + 2 < NCH)
            def _():
                start_chunk(c + 2, b)

        def pair_body(pair, carry):
            for b in range(2):
                _chunk(pair * 2 + b, b)
            return carry

        lax.fori_loop(0, NCH // 2, pair_body, 0)
        pltpu.sync_copy(ov, out_hbm.at[pl.ds(base, PW)])

    return sc_duelqa


_SC = _make_sc()


def kernel(x, a):
    a32 = a.reshape(-1).astype(jnp.int32)
    partial = _SC(x, a32)
    return (partial + x[:, NADV])[:, None]


# speedup vs baseline: 1.0340x; 1.0340x over previous
import functools

import jax
import jax.numpy as jnp
from jax import lax
from jax.experimental import pallas as pl
from jax.experimental.pallas import tpu as pltpu
from jax.experimental.pallas import tpu_sc as plsc

B = 16384
C = 1001
NADV = 1000
S = 1.0 / NADV

RT = 8192
R = 2048

NC, NS, L = 2, 16, 16
NW = NC * NS
BSC = B - RT
PW = BSC // NW
CH = 32
NCH = PW // CH
NG = CH // L


def _tc_body(x_ref, a_ref, o_ref):
    xb = x_ref[...]
    av = a_ref[...]
    cols = lax.broadcasted_iota(jnp.int32, (R, C), 1)
    sel = jnp.where(cols == av, xb, 0.0)
    wcol = lax.broadcasted_iota(jnp.int32, (C, 1), 0)
    w = jnp.where(wcol == NADV, jnp.float32(1.0), jnp.float32(-S))
    ones = jnp.full((C, 1), jnp.float32(1.0))
    d = jnp.dot(xb, w, preferred_element_type=jnp.float32)
    g = jnp.dot(sel, ones, preferred_element_type=jnp.float32)
    o_ref[...] = d + g


def _make_sc():
    mesh = plsc.VectorSubcoreMesh(core_axis_name="c", subcore_axis_name="s")

    @functools.partial(
        pl.kernel,
        out_type=jax.ShapeDtypeStruct((BSC,), jnp.float32),
        mesh=mesh,
        compiler_params=pltpu.CompilerParams(needs_layout_passes=False),
        scratch_types=[
            pltpu.VMEM((CH, C), jnp.float32),
            pltpu.VMEM((CH, C), jnp.float32),
            pltpu.VMEM((PW,), jnp.int32),
            pltpu.VMEM((PW,), jnp.float32),
            pltpu.SemaphoreType.DMA((2,)),
            pltpu.SemaphoreType.DMA,
        ],
    )
    def sc_duelqa(x_hbm, a_hbm, out_hbm, xv0, xv1, av, ov, sems, asem):
        wid = lax.axis_index("s") * NC + lax.axis_index("c")
        obase = wid * PW
        base = RT + obase
        pltpu.async_copy(a_hbm.at[pl.ds(obase, PW)], av, asem).wait()
        lane = lax.iota(jnp.int32, L)
        zero16 = jnp.zeros((L,), jnp.float32)
        tailm = (lane >= 8).astype(jnp.float32)

        def cp(c, b):
            return pltpu.make_async_copy(
                x_hbm.at[pl.ds(base + c * CH, CH), :],
                xv0 if b == 0 else xv1,
                sems.at[b],
            )

        cp(0, 0).start()
        cp(1, 1).start()

        def _chunk(c, b):
            cp(c, b).wait()
            xb = xv0 if b == 0 else xv1
            for g in range(NG):
                lrows = g * L + lane

                def row_body(r, tvec):
                    row = g * L + r
                    accs = [zero16, zero16, zero16, zero16]
                    for j in range(62):
                        accs[j % 4] = accs[j % 4] + xb[row, pl.ds(j * L, L)]
                    tail = xb[row, pl.ds(984, L)] * tailm
                    acc = (accs[0] + accs[1]) + (accs[2] + accs[3]) + tail
                    t = jnp.sum(acc)
                    oh = (lane == r).astype(jnp.float32)
                    return tvec + t * oh

                tvec = lax.fori_loop(0, L, row_body, zero16)
                off = c * CH + g * L
                a16 = av[pl.ds(off, L)]
                gv = plsc.load_gather(xb, [lrows, a16])
                ov[pl.ds(off, L)] = gv - tvec * jnp.float32(S)

            @pl.when(c + 2 < NCH)
            def _():
                cp(c + 2, b).start()

        def pair_body(pair, carry):
            for b in range(2):
                _chunk(pair * 2 + b, b)
            return carry

        lax.fori_loop(0, NCH // 2, pair_body, 0)
        pltpu.sync_copy(ov, out_hbm.at[pl.ds(obase, PW)])

    return sc_duelqa


_SC = _make_sc()


def kernel(x, a):
    a32 = a.astype(jnp.int32)
    sc_part = _SC(x, a32.reshape(-1)[RT:])
    tc_part = pl.pallas_call(
        _tc_body,
        grid=(RT // R,),
        in_specs=[
            pl.BlockSpec((R, C), lambda i: (i, 0)),
            pl.BlockSpec((R, 1), lambda i: (i, 0)),
        ],
        out_specs=pl.BlockSpec((R, 1), lambda i: (i, 0)),
        out_shape=jax.ShapeDtypeStruct((RT, 1), jnp.float32),
    )(x, a32)
    sc_full = (sc_part + x[RT:, NADV])[:, None]
    return jnp.concatenate([tc_part, sc_full], axis=0)


# speedup vs baseline: 1.0574x; 1.0226x over previous
import functools

import jax
import jax.numpy as jnp
from jax import lax
from jax.experimental import pallas as pl
from jax.experimental.pallas import tpu as pltpu
from jax.experimental.pallas import tpu_sc as plsc

B = 16384
C = 1001
NADV = 1000
S = 1.0 / NADV

NC, NS, L = 2, 16, 16
NW = NC * NS
PW = B // NW
CH = 32
NCH = PW // CH
NG = CH // L


def _make_sc():
    mesh = plsc.VectorSubcoreMesh(core_axis_name="c", subcore_axis_name="s")

    @functools.partial(
        pl.kernel,
        out_type=jax.ShapeDtypeStruct((B,), jnp.float32),
        mesh=mesh,
        compiler_params=pltpu.CompilerParams(needs_layout_passes=False),
        scratch_types=[
            pltpu.VMEM((CH, C), jnp.float32),
            pltpu.VMEM((CH, C), jnp.float32),
            pltpu.VMEM((PW,), jnp.int32),
            pltpu.VMEM((PW,), jnp.float32),
            pltpu.SemaphoreType.DMA((2,)),
            pltpu.SemaphoreType.DMA,
        ],
    )
    def sc_duelqa(x_hbm, a_hbm, out_hbm, xv0, xv1, av, ov, sems, asem):
        wid = lax.axis_index("s") * NC + lax.axis_index("c")
        base = wid * PW
        pltpu.async_copy(a_hbm.at[pl.ds(base, PW)], av, asem).wait()
        lane = lax.iota(jnp.int32, L)
        zero16 = jnp.zeros((L,), jnp.float32)
        tailm = (lane >= 8).astype(jnp.float32)

        def cp(c, b):
            return pltpu.make_async_copy(
                x_hbm.at[pl.ds(base + c * CH, CH), :],
                xv0 if b == 0 else xv1,
                sems.at[b],
            )

        cp(0, 0).start()
        cp(1, 1).start()

        def _chunk(c, b):
            cp(c, b).wait()
            xb = xv0 if b == 0 else xv1
            for g in range(NG):
                lrows = g * L + lane

                def row_body(r, tvec):
                    row = g * L + r
                    accs = [zero16, zero16, zero16, zero16]
                    for j in range(62):
                        accs[j % 4] = accs[j % 4] + xb[row, pl.ds(j * L, L)]
                    tail = xb[row, pl.ds(984, L)] * tailm
                    acc = (accs[0] + accs[1]) + (accs[2] + accs[3]) + tail
                    t = jnp.sum(acc)
                    oh = (lane == r).astype(jnp.float32)
                    return tvec + t * oh

                tvec = lax.fori_loop(0, L, row_body, zero16)
                off = c * CH + g * L
                a16 = av[pl.ds(off, L)]
                gv = plsc.load_gather(xb, [lrows, a16])
                ov[pl.ds(off, L)] = gv - tvec * jnp.float32(S)

            @pl.when(c + 2 < NCH)
            def _():
                cp(c + 2, b).start()

        def pair_body(pair, carry):
            for b in range(2):
                _chunk(pair * 2 + b, b)
            return carry

        lax.fori_loop(0, NCH // 2, pair_body, 0)
        pltpu.sync_copy(ov, out_hbm.at[pl.ds(base, PW)])

    return sc_duelqa


_SC = _make_sc()


def kernel(x, a):
    a32 = a.reshape(-1).astype(jnp.int32)
    partial = _SC(x, a32)
    return (partial + x[:, NADV])[:, None]
